# baseline (speedup 1.0000x reference)
import jax
import jax.numpy as jnp
from jax import lax
from jax.experimental import pallas as pl
from jax.experimental.pallas import tpu as pltpu
from jax.experimental.pallas import tpu_sc as plsc

N = 10000
E = 320000
D = 128
NCLS = 40
NC = 2
NS = 16
NW = NC * NS
EPT = E // NW
CH = 80
NSEG = 16
NSEGS = 8
NCHP = NSEG * NSEGS
EP = NW * NCHP * CH
ND = N + 8
CHD = 80
NCHD = EPT // CHD
NSW = 10
ROWS_PT = N // NSW
ZR = 40

_mesh = plsc.VectorSubcoreMesh(core_axis_name="c", subcore_axis_name="s")


def _fill_f32(ref, rows, cols, value):
    @pl.loop(0, rows)
    def _(r):
        @pl.loop(0, cols, step=16)
        def _(cc):
            ref[r, pl.ds(cc, 16)] = jnp.full((16,), value, jnp.float32)


def _deg_body(dst3_hbm, out_hbm, didx, ones, zbuf, deg_sh, sem):
    c = lax.axis_index("c")
    s = lax.axis_index("s")
    wid = c * NS + s
    pltpu.async_copy(dst3_hbm.at[wid], didx, sem)

    @pl.when(s == 0)
    def _():
        @pl.loop(0, 2000 // 16)
        def _(i):
            zbuf[pl.ds(i * 16, 16)] = jnp.zeros((16,), jnp.float32)
        @pl.loop(0, N // 2000)
        def _(j):
            pltpu.sync_copy(zbuf, deg_sh.at[pl.ds(j * 2000, 2000)])

    @pl.loop(0, CHD // 16)
    def _(i):
        ones[pl.ds(i * 16, 16)] = jnp.ones((16,), jnp.float32)
    pltpu.make_async_copy(dst3_hbm.at[wid], didx, sem).wait()

    plsc.subcore_barrier()

    @pl.loop(0, NCHD)
    def _(j):
        pltpu.async_copy(ones, deg_sh.at[didx.at[j]], sem, add=True)
    @pl.loop(0, NCHD)
    def _(j):
        pltpu.make_async_copy(ones, deg_sh.at[didx.at[j]], sem).wait()

    plsc.subcore_barrier()

    @pl.when(s == 0)
    def _():
        pltpu.sync_copy(deg_sh, out_hbm.at[c])


def _sc_deg(dst4d):
    k = pl.kernel(
        _deg_body,
        out_type=jax.ShapeDtypeStruct((NC, N), jnp.float32),
        mesh=_mesh,
        scratch_types=[
            pltpu.VMEM((NCHD, CHD), jnp.int32),
            pltpu.VMEM((CHD,), jnp.float32),
            pltpu.VMEM((2000,), jnp.float32),
            pltpu.VMEM_SHARED((N,), jnp.float32),
            pltpu.SemaphoreType.DMA,
        ],
    )
    return k(dst4d)


def _agg_body(hn_hbm, src3_hbm, dst3_hbm, out_hbm, sA, dA, sB, dB,
              r0, r1, r2, r3, acc_sh, sg0, sg1, sg2, sg3,
              ss0, ss1, ss2, ss3, si):
    c = lax.axis_index("c")
    s = lax.axis_index("s")
    wid = c * NS + s
    rows = (r0, r1, r2, r3)
    sg = (sg0, sg1, sg2, sg3)
    ss = (ss0, ss1, ss2, ss3)

    def load_seg(q, sT, dT):
        pltpu.async_copy(src3_hbm.at[wid, pl.ds(q * NSEG, NSEG)], sT, si)
        pltpu.async_copy(dst3_hbm.at[wid, pl.ds(q * NSEG, NSEG)], dT, si)

    def wait_seg(sT, dT):
        pltpu.make_async_copy(src3_hbm.at[wid, pl.ds(0, NSEG)], sT, si).wait()
        pltpu.make_async_copy(dst3_hbm.at[wid, pl.ds(0, NSEG)], dT, si).wait()

    load_seg(0, sA, dA)

    @pl.when(s < NSW)
    def _():
        _fill_f32(r0, ZR, D, 0.0)
        @pl.loop(0, ROWS_PT // ZR)
        def _(j):
            pltpu.sync_copy(r0.at[pl.ds(0, ZR)],
                            acc_sh.at[pl.ds(s * ROWS_PT + j * ZR, ZR)])
        @pl.when(s == 0)
        def _():
            pltpu.sync_copy(r0.at[pl.ds(0, ND - N)],
                            acc_sh.at[pl.ds(N, ND - N)])

    wait_seg(sA, dA)
    plsc.subcore_barrier()

    def start_gather(sT, k, b):
        pltpu.async_copy(hn_hbm.at[sT.at[k]], rows[b], sg[b])

    def wait_gather(sT, k, b):
        pltpu.make_async_copy(hn_hbm.at[sT.at[k]], rows[b], sg[b]).wait()

    def start_scatter(dT, k, b):
        pltpu.async_copy(rows[b], acc_sh.at[dT.at[k]], ss[b], add=True)

    def wait_scatter(dT, k, b):
        pltpu.make_async_copy(rows[b], acc_sh.at[dT.at[k]], ss[b]).wait()

    def do_segment(sT, dT):
        for b in range(3):
            start_gather(sT, b, b)
        wait_gather(sT, 0, 0)
        start_scatter(dT, 0, 0)
        start_gather(sT, 3, 3)

        @pl.loop(1, 13, step=4)
        def _(k):
            for cc in range(4):
                bb = (1 + cc) % 4
                wait_gather(sT, k + cc, bb)
                start_scatter(dT, k + cc, bb)
                wait_scatter(dT, k + cc - 1, cc % 4)
                start_gather(sT, k + cc + 3, cc % 4)

        for j in range(13, 16):
            wait_gather(sT, j, j % 4)
            start_scatter(dT, j, j % 4)
            wait_scatter(dT, j - 1, (j - 1) % 4)
        wait_scatter(dT, 15, 3)

    @pl.loop(0, NSEGS - 2, step=2)
    def _(qq):
        load_seg(qq + 1, sB, dB)
        do_segment(sA, dA)
        wait_seg(sB, dB)
        load_seg(qq + 2, sA, dA)
        do_segment(sB, dB)
        wait_seg(sA, dA)
    load_seg(NSEGS - 1, sB, dB)
    do_segment(sA, dA)
    wait_seg(sB, dB)
    do_segment(sB, dB)

    plsc.subcore_barrier()

    @pl.when(s < NSW)
    def _():
        pltpu.sync_copy(
            acc_sh.at[pl.ds(s * ROWS_PT, ROWS_PT)],
            out_hbm.at[c, pl.ds(s * ROWS_PT, ROWS_PT)],
        )


def _sc_agg(hn, src3d, dst3d):
    k = pl.kernel(
        _agg_body,
        out_type=jax.ShapeDtypeStruct((NC, N, D), jnp.float32),
        mesh=_mesh,
        scratch_types=[
            pltpu.VMEM((NSEG, CH), jnp.int32),
            pltpu.VMEM((NSEG, CH), jnp.int32),
            pltpu.VMEM((NSEG, CH), jnp.int32),
            pltpu.VMEM((NSEG, CH), jnp.int32),
            pltpu.VMEM((CH, D), jnp.float32),
            pltpu.VMEM((CH, D), jnp.float32),
            pltpu.VMEM((CH, D), jnp.float32),
            pltpu.VMEM((CH, D), jnp.float32),
            pltpu.VMEM_SHARED((ND, D), jnp.float32),
            pltpu.SemaphoreType.DMA,
            pltpu.SemaphoreType.DMA,
            pltpu.SemaphoreType.DMA,
            pltpu.SemaphoreType.DMA,
            pltpu.SemaphoreType.DMA,
            pltpu.SemaphoreType.DMA,
            pltpu.SemaphoreType.DMA,
            pltpu.SemaphoreType.DMA,
            pltpu.SemaphoreType.DMA,
        ],
    )
    return k(hn, src3d, dst3d)


_PREC = lax.Precision.HIGHEST


def _norm_from_deg(deg_ref):
    d = deg_ref[0] + deg_ref[1]
    return jnp.where(d > 0, lax.rsqrt(jnp.maximum(d, 1.0)), 0.0)


def _tc_in_body(deg_ref, x_ref, w_ref, o_ref):
    norm = _norm_from_deg(deg_ref)
    h = jnp.dot(x_ref[...], w_ref[...], preferred_element_type=jnp.float32,
                precision=_PREC)
    o_ref[...] = jnp.maximum(h, 0.0) * norm


def _tc_layer_body(deg_ref, p_ref, w_ref, o_ref):
    norm = _norm_from_deg(deg_ref)
    a = (p_ref[0] + p_ref[1]) * norm
    h = jnp.dot(a, w_ref[...], preferred_element_type=jnp.float32,
                precision=_PREC)
    o_ref[...] = jnp.maximum(h, 0.0) * norm


def _tc_out_body(deg_ref, p_ref, wg_ref, wo_ref, o_ref):
    norm = _norm_from_deg(deg_ref)
    a = (p_ref[0] + p_ref[1]) * norm
    h = jnp.maximum(
        jnp.dot(a, wg_ref[...], preferred_element_type=jnp.float32,
                precision=_PREC), 0.0)
    o_ref[...] = jnp.dot(h, wo_ref[...], preferred_element_type=jnp.float32,
                         precision=_PREC)


_B = 2000


def _deg_spec():
    return pl.BlockSpec((NC, _B, 1), lambda i: (0, i, 0))


def _w_spec(r, c):
    return pl.BlockSpec((r, c), lambda i: (0, 0))


def _tc_in(degp, x, w):
    return pl.pallas_call(
        _tc_in_body,
        grid=(N // _B,),
        in_specs=[_deg_spec(),
                  pl.BlockSpec((_B, D), lambda i: (i, 0)),
                  _w_spec(D, D)],
        out_specs=pl.BlockSpec((_B, D), lambda i: (i, 0)),
        out_shape=jax.ShapeDtypeStruct((N, D), jnp.float32),
    )(degp, x, w)


def _tc_layer(degp, p, w):
    return pl.pallas_call(
        _tc_layer_body,
        grid=(N // _B,),
        in_specs=[_deg_spec(),
                  pl.BlockSpec((NC, _B, D), lambda i: (0, i, 0)),
                  _w_spec(D, D)],
        out_specs=pl.BlockSpec((_B, D), lambda i: (i, 0)),
        out_shape=jax.ShapeDtypeStruct((N, D), jnp.float32),
    )(degp, p, w)


def _tc_out(degp, p, wg, wo):
    return pl.pallas_call(
        _tc_out_body,
        grid=(N // _B,),
        in_specs=[_deg_spec(),
                  pl.BlockSpec((NC, _B, D), lambda i: (0, i, 0)),
                  _w_spec(D, D),
                  _w_spec(D, NCLS)],
        out_specs=pl.BlockSpec((_B, NCLS), lambda i: (i, 0)),
        out_shape=jax.ShapeDtypeStruct((N, NCLS), jnp.float32),
    )(degp, p, wg, wo)


def kernel(features, edge_index, W_in, W_g0, W_g1, W_out):
    src = edge_index[0].astype(jnp.int32)
    dst = edge_index[1].astype(jnp.int32)
    pad = EP - E
    src3d = jnp.concatenate(
        [src, jnp.zeros((pad,), jnp.int32)]).reshape(NW, NCHP, CH)
    dst3d = jnp.concatenate(
        [dst, N + (jnp.arange(pad, dtype=jnp.int32) % (ND - N))]
    ).reshape(NW, NCHP, CH)
    dst4d = dst.reshape(NW, NCHD, CHD)

    degp = _sc_deg(dst4d)
    degp3 = degp.reshape(NC, N, 1)

    hn0 = _tc_in(degp3, features, W_in)
    p0 = _sc_agg(hn0, src3d, dst3d)
    hn1 = _tc_layer(degp3, p0, W_g0)
    p1 = _sc_agg(hn1, src3d, dst3d)
    out = _tc_out(degp3, p1, W_g1, W_out)
    return out


# speedup vs baseline: 2.8411x; 2.8411x over previous
import jax
import jax.numpy as jnp
from jax import lax
from jax.experimental import pallas as pl
from jax.experimental.pallas import tpu as pltpu
from jax.experimental.pallas import tpu_sc as plsc

N = 10000
E = 320000
D = 128
NCLS = 40
NC = 2
NS = 16
NW = NC * NS
EPT = E // NW
CH = 80
NSEG = 16
NSEGS = 8
NCHP = NSEG * NSEGS
EP = NW * NCHP * CH
HNP = N + 8
CHD = 80
NCHD = EPT // CHD
NSW = 10
ROWS_PT = N // NSW
ZR = 40

_mesh = plsc.VectorSubcoreMesh(core_axis_name="c", subcore_axis_name="s")


def _fill_f32(ref, rows, cols, value):
    @pl.loop(0, rows)
    def _(r):
        @pl.loop(0, cols, step=16)
        def _(cc):
            ref[r, pl.ds(cc, 16)] = jnp.full((16,), value, jnp.float32)


def _deg_body(dst3_hbm, out_hbm, didx, ones, zbuf, deg_sh, sem):
    c = lax.axis_index("c")
    s = lax.axis_index("s")
    wid = c * NS + s
    pltpu.async_copy(dst3_hbm.at[wid], didx, sem)

    @pl.when(s == 0)
    def _():
        @pl.loop(0, 2000 // 16)
        def _(i):
            zbuf[pl.ds(i * 16, 16)] = jnp.zeros((16,), jnp.float32)
        @pl.loop(0, N // 2000)
        def _(j):
            pltpu.sync_copy(zbuf, deg_sh.at[pl.ds(j * 2000, 2000)])

    @pl.loop(0, CHD // 16)
    def _(i):
        ones[pl.ds(i * 16, 16)] = jnp.ones((16,), jnp.float32)
    pltpu.make_async_copy(dst3_hbm.at[wid], didx, sem).wait()

    plsc.subcore_barrier()

    @pl.loop(0, NCHD)
    def _(j):
        pltpu.async_copy(ones, deg_sh.at[didx.at[j]], sem, add=True)
    @pl.loop(0, NCHD)
    def _(j):
        pltpu.make_async_copy(ones, deg_sh.at[didx.at[j]], sem).wait()

    plsc.subcore_barrier()

    @pl.when(s == 0)
    def _():
        pltpu.sync_copy(deg_sh, out_hbm.at[c])


def _sc_deg(dst4d):
    k = pl.kernel(
        _deg_body,
        out_type=jax.ShapeDtypeStruct((NC, N), jnp.float32),
        mesh=_mesh,
        scratch_types=[
            pltpu.VMEM((NCHD, CHD), jnp.int32),
            pltpu.VMEM((CHD,), jnp.float32),
            pltpu.VMEM((2000,), jnp.float32),
            pltpu.VMEM_SHARED((N,), jnp.float32),
            pltpu.SemaphoreType.DMA,
        ],
    )
    return k(dst4d)


def _agg_body(hn_hbm, src3_hbm, dst3_hbm, out_hbm, sA, dA, sB, dB,
              r0, r1, r2, r3, acc_sh, sg0, sg1, sg2, sg3,
              ss0, ss1, ss2, ss3, si):
    c = lax.axis_index("c")
    s = lax.axis_index("s")
    wid = c * NS + s
    rows = (r0, r1, r2, r3)
    sg = (sg0, sg1, sg2, sg3)
    ss = (ss0, ss1, ss2, ss3)

    def load_seg(q, sT, dT):
        pltpu.async_copy(src3_hbm.at[wid, pl.ds(q * NSEG, NSEG)], sT, si)
        pltpu.async_copy(dst3_hbm.at[wid, pl.ds(q * NSEG, NSEG)], dT, si)

    def wait_seg(sT, dT):
        pltpu.make_async_copy(src3_hbm.at[wid, pl.ds(0, NSEG)], sT, si).wait()
        pltpu.make_async_copy(dst3_hbm.at[wid, pl.ds(0, NSEG)], dT, si).wait()

    load_seg(0, sA, dA)

    @pl.when(s < NSW)
    def _():
        _fill_f32(r0, ZR, D, 0.0)
        @pl.loop(0, ROWS_PT // ZR)
        def _(j):
            pltpu.sync_copy(r0.at[pl.ds(0, ZR)],
                            acc_sh.at[pl.ds(s * ROWS_PT + j * ZR, ZR)])
    wait_seg(sA, dA)
    plsc.subcore_barrier()

    def start_gather(sT, k, b):
        pltpu.async_copy(hn_hbm.at[sT.at[k]], rows[b], sg[b])

    def wait_gather(sT, k, b):
        pltpu.make_async_copy(hn_hbm.at[sT.at[k]], rows[b], sg[b]).wait()

    def start_scatter(dT, k, b):
        pltpu.async_copy(rows[b], acc_sh.at[dT.at[k]], ss[b], add=True)

    def wait_scatter(dT, k, b):
        pltpu.make_async_copy(rows[b], acc_sh.at[dT.at[k]], ss[b]).wait()

    def do_segment(sT, dT):
        for b in range(3):
            start_gather(sT, b, b)
        wait_gather(sT, 0, 0)
        start_scatter(dT, 0, 0)
        start_gather(sT, 3, 3)

        @pl.loop(1, 13, step=4)
        def _(k):
            for cc in range(4):
                bb = (1 + cc) % 4
                wait_gather(sT, k + cc, bb)
                start_scatter(dT, k + cc, bb)
                wait_scatter(dT, k + cc - 1, cc % 4)
                start_gather(sT, k + cc + 3, cc % 4)

        for j in range(13, 16):
            wait_gather(sT, j, j % 4)
            start_scatter(dT, j, j % 4)
            wait_scatter(dT, j - 1, (j - 1) % 4)
        wait_scatter(dT, 15, 3)

    @pl.loop(0, NSEGS - 2, step=2)
    def _(qq):
        load_seg(qq + 1, sB, dB)
        do_segment(sA, dA)
        wait_seg(sB, dB)
        load_seg(qq + 2, sA, dA)
        do_segment(sB, dB)
        wait_seg(sA, dA)
    load_seg(NSEGS - 1, sB, dB)
    do_segment(sA, dA)
    wait_seg(sB, dB)
    do_segment(sB, dB)

    plsc.subcore_barrier()

    @pl.when(s < NSW)
    def _():
        pltpu.sync_copy(
            acc_sh.at[pl.ds(s * ROWS_PT, ROWS_PT)],
            out_hbm.at[c, pl.ds(s * ROWS_PT, ROWS_PT)],
        )


def _sc_agg(hn, src3d, dst3d):
    k = pl.kernel(
        _agg_body,
        out_type=jax.ShapeDtypeStruct((NC, N, D), jnp.float32),
        mesh=_mesh,
        scratch_types=[
            pltpu.VMEM((NSEG, CH), jnp.int32),
            pltpu.VMEM((NSEG, CH), jnp.int32),
            pltpu.VMEM((NSEG, CH), jnp.int32),
            pltpu.VMEM((NSEG, CH), jnp.int32),
            pltpu.VMEM((CH, D), jnp.float32),
            pltpu.VMEM((CH, D), jnp.float32),
            pltpu.VMEM((CH, D), jnp.float32),
            pltpu.VMEM((CH, D), jnp.float32),
            pltpu.VMEM_SHARED((N, D), jnp.float32),
            pltpu.SemaphoreType.DMA,
            pltpu.SemaphoreType.DMA,
            pltpu.SemaphoreType.DMA,
            pltpu.SemaphoreType.DMA,
            pltpu.SemaphoreType.DMA,
            pltpu.SemaphoreType.DMA,
            pltpu.SemaphoreType.DMA,
            pltpu.SemaphoreType.DMA,
            pltpu.SemaphoreType.DMA,
        ],
    )
    return k(hn, src3d, dst3d)


_PREC = lax.Precision.HIGHEST


def _norm_from_deg(deg_ref):
    d = deg_ref[0] + deg_ref[1]
    return jnp.where(d > 0, lax.rsqrt(jnp.maximum(d, 1.0)), 0.0)


def _tc_in_body(deg_ref, x_ref, w_ref, o_ref):
    norm = _norm_from_deg(deg_ref)
    h = jnp.dot(x_ref[...], w_ref[...], preferred_element_type=jnp.float32,
                precision=_PREC)
    o_ref[...] = jnp.maximum(h, 0.0) * norm


def _tc_layer_body(deg_ref, p_ref, w_ref, o_ref):
    norm = _norm_from_deg(deg_ref)
    a = (p_ref[0] + p_ref[1]) * norm
    h = jnp.dot(a, w_ref[...], preferred_element_type=jnp.float32,
                precision=_PREC)
    o_ref[...] = jnp.maximum(h, 0.0) * norm


def _tc_out_body(deg_ref, p_ref, wg_ref, wo_ref, o_ref):
    norm = _norm_from_deg(deg_ref)
    a = (p_ref[0] + p_ref[1]) * norm
    h = jnp.maximum(
        jnp.dot(a, wg_ref[...], preferred_element_type=jnp.float32,
                precision=_PREC), 0.0)
    o_ref[...] = jnp.dot(h, wo_ref[...], preferred_element_type=jnp.float32,
                         precision=_PREC)


_B = 2000


def _deg_spec():
    return pl.BlockSpec((NC, _B, 1), lambda i: (0, i, 0))


def _w_spec(r, c):
    return pl.BlockSpec((r, c), lambda i: (0, 0))


def _tc_in(degp, x, w):
    return pl.pallas_call(
        _tc_in_body,
        grid=(N // _B,),
        in_specs=[_deg_spec(),
                  pl.BlockSpec((_B, D), lambda i: (i, 0)),
                  _w_spec(D, D)],
        out_specs=pl.BlockSpec((_B, D), lambda i: (i, 0)),
        out_shape=jax.ShapeDtypeStruct((N, D), jnp.float32),
    )(degp, x, w)


def _tc_layer(degp, p, w):
    return pl.pallas_call(
        _tc_layer_body,
        grid=(N // _B,),
        in_specs=[_deg_spec(),
                  pl.BlockSpec((NC, _B, D), lambda i: (0, i, 0)),
                  _w_spec(D, D)],
        out_specs=pl.BlockSpec((_B, D), lambda i: (i, 0)),
        out_shape=jax.ShapeDtypeStruct((N, D), jnp.float32),
    )(degp, p, w)


def _tc_out(degp, p, wg, wo):
    return pl.pallas_call(
        _tc_out_body,
        grid=(N // _B,),
        in_specs=[_deg_spec(),
                  pl.BlockSpec((NC, _B, D), lambda i: (0, i, 0)),
                  _w_spec(D, D),
                  _w_spec(D, NCLS)],
        out_specs=pl.BlockSpec((_B, NCLS), lambda i: (i, 0)),
        out_shape=jax.ShapeDtypeStruct((N, NCLS), jnp.float32),
    )(degp, p, wg, wo)


def kernel(features, edge_index, W_in, W_g0, W_g1, W_out):
    src = edge_index[0].astype(jnp.int32)
    dst = edge_index[1].astype(jnp.int32)
    pad = EP - E
    src3d = jnp.concatenate(
        [src, N + (jnp.arange(pad, dtype=jnp.int32) % (HNP - N))]
    ).reshape(NW, NCHP, CH)
    dst3d = jnp.concatenate(
        [dst, jnp.arange(pad, dtype=jnp.int32) % N]).reshape(NW, NCHP, CH)
    dst4d = dst.reshape(NW, NCHD, CHD)

    degp = _sc_deg(dst4d)
    degp3 = degp.reshape(NC, N, 1)

    zpad = jnp.zeros((HNP - N, D), jnp.float32)
    hn0 = _tc_in(degp3, features, W_in)
    p0 = _sc_agg(jnp.concatenate([hn0, zpad]), src3d, dst3d)
    hn1 = _tc_layer(degp3, p0, W_g0)
    p1 = _sc_agg(jnp.concatenate([hn1, zpad]), src3d, dst3d)
    out = _tc_out(degp3, p1, W_g1, W_out)
    return out
